# baseline (speedup 1.0000x reference)
import jax
import jax.numpy as jnp
from jax.experimental import pallas as pl
from jax.experimental.pallas import tpu as pltpu

_CH = 512
_NBUF = 10


def _body(a_hbm, f_ref, o_ref, buf, sems):
    B, M, K = a_hbm.shape
    cpb = M // _CH
    total = B * cpb

    def copy(c):
        b, r = divmod(c, cpb)
        return pltpu.make_async_copy(
            a_hbm.at[b, pl.ds(r * _CH, _CH), :],
            buf.at[c % _NBUF],
            sems.at[c % _NBUF],
        )

    for c in range(min(_NBUF, total)):
        copy(c).start()

    for c in range(total):
        b, r = divmod(c, cpb)
        copy(c).wait()
        o_ref[b, r * _CH:(r + 1) * _CH, :] = jax.lax.dot_general(
            buf[c % _NBUF], f_ref[b], (((1,), (0,)), ((), ())),
            precision=jax.lax.Precision.DEFAULT,
            preferred_element_type=jnp.float32)
        if c + _NBUF < total:
            copy(c + _NBUF).start()


def kernel(features, A):
    B, M, K = A.shape
    N = features.shape[-1]
    return pl.pallas_call(
        _body,
        in_specs=[
            pl.BlockSpec(memory_space=pltpu.MemorySpace.HBM),
            pl.BlockSpec(memory_space=pltpu.MemorySpace.VMEM),
        ],
        out_specs=pl.BlockSpec(memory_space=pltpu.MemorySpace.VMEM),
        out_shape=jax.ShapeDtypeStruct((B, M, N), jnp.float32),
        scratch_shapes=[
            pltpu.VMEM((_NBUF, _CH, K), jnp.float32),
            pltpu.SemaphoreType.DMA((_NBUF,)),
        ],
    )(A, features)


# speedup vs baseline: 1.0014x; 1.0014x over previous
import jax
import jax.numpy as jnp
from jax.experimental import pallas as pl
from jax.experimental.pallas import tpu as pltpu

_CH = 512
_NBUF = 10


def _body(a_hbm, f_ref, o_ref, buf, sems):
    B, M, K = a_hbm.shape
    cpb = M // _CH
    total = B * cpb

    def copy(c):
        b, r = divmod(c, cpb)
        return pltpu.make_async_copy(
            a_hbm.at[b, pl.ds(r * _CH, _CH), :],
            buf.at[c % _NBUF],
            sems.at[c % _NBUF],
        )

    for c in range(min(_NBUF - 1, total)):
        copy(c).start()

    for c in range(total):
        b, r = divmod(c, cpb)
        if _NBUF - 1 <= c + _NBUF - 1 < total:
            copy(c + _NBUF - 1).start()
        copy(c).wait()
        o_ref[b, r * _CH:(r + 1) * _CH, :] = jax.lax.dot_general(
            buf[c % _NBUF], f_ref[b], (((1,), (0,)), ((), ())),
            precision=jax.lax.Precision.DEFAULT,
            preferred_element_type=jnp.float32)


def kernel(features, A):
    B, M, K = A.shape
    N = features.shape[-1]
    return pl.pallas_call(
        _body,
        in_specs=[
            pl.BlockSpec(memory_space=pltpu.MemorySpace.HBM),
            pl.BlockSpec(memory_space=pltpu.MemorySpace.VMEM),
        ],
        out_specs=pl.BlockSpec(memory_space=pltpu.MemorySpace.VMEM),
        out_shape=jax.ShapeDtypeStruct((B, M, N), jnp.float32),
        scratch_shapes=[
            pltpu.VMEM((_NBUF, _CH, K), jnp.float32),
            pltpu.SemaphoreType.DMA((_NBUF,)),
        ],
    )(A, features)


# speedup vs baseline: 1.0408x; 1.0394x over previous
import jax
import jax.numpy as jnp
from jax.experimental import pallas as pl
from jax.experimental.pallas import tpu as pltpu

_NS = 8
_BMS = 256


def _bmm_kernel(f_ref, *refs):
    a_refs, o_ref = refs[:_NS], refs[_NS]
    f = f_ref[0]
    for j in range(_NS):
        o_ref[0, j * _BMS:(j + 1) * _BMS, :] = jnp.dot(
            a_refs[j][0], f, preferred_element_type=jnp.float32)


def kernel(features, A):
    B, M, K = A.shape
    N = features.shape[-1]
    bm = _NS * _BMS
    a_specs = [
        pl.BlockSpec((1, _BMS, K), lambda b, i, j=j: (b, i * _NS + j, 0))
        for j in range(_NS)
    ]
    return pl.pallas_call(
        _bmm_kernel,
        grid=(B, M // bm),
        in_specs=[pl.BlockSpec((1, K, N), lambda b, i: (b, 0, 0))] + a_specs,
        out_specs=pl.BlockSpec((1, bm, N), lambda b, i: (b, i, 0)),
        out_shape=jax.ShapeDtypeStruct((B, M, N), jnp.float32),
        compiler_params=pltpu.CompilerParams(
            dimension_semantics=("parallel", "parallel"),
        ),
    )(features, *([A] * _NS))
